# Initial kernel scaffold; baseline (speedup 1.0000x reference)
import jax
import jax.numpy as jnp
from jax.experimental import pallas as pl


def kernel(hidden_states, router_weight, gate_up_proj, down_proj, gate_w, up_w, down_w, shared_gate_w):
    raise NotImplementedError("write your pallas kernel here")


# speedup vs baseline: 1.1417x; 1.1417x over previous
import functools

import jax
import jax.numpy as jnp
from jax.experimental import pallas as pl
from jax.experimental.pallas import tpu as pltpu

_S, _B, _D = 2048, 1, 1024
_E, _TOPK = 8, 2
_I = 512
_SI = 512
_T = _S * _B
_TB = 256
_NTB = _T // _TB


def _moe_body(x_ref, wr_ref, gup_ref, down_ref, gw_ref, uw_ref, dw_ref, sg_ref,
              out_ref, probs_ref, acc_ref):
    e = pl.program_id(0)
    tb = pl.program_id(1)

    x = x_ref[...]

    logits = jax.lax.dot_general(x, wr_ref[...], (((1,), (1,)), ((), ())),
                                 preferred_element_type=jnp.float32)
    m = jnp.max(logits, axis=-1, keepdims=True)
    ex = jnp.exp(logits - m)
    probs = ex / jnp.sum(ex, axis=-1, keepdims=True)
    probs_ref[...] = probs

    iota_e = jax.lax.broadcasted_iota(jnp.int32, (_TB, _E), 1)
    i1 = jnp.argmax(probs, axis=-1)[:, None]
    oh1 = (iota_e == i1)
    m1 = jnp.max(probs, axis=-1, keepdims=True)
    masked = jnp.where(oh1, -jnp.inf, probs)
    i2 = jnp.argmax(masked, axis=-1)[:, None]
    oh2 = (iota_e == i2)
    m2 = jnp.max(masked, axis=-1, keepdims=True)
    denom = m1 + m2 + 1e-9
    w_te = (jnp.where(oh1, m1, 0.0) + jnp.where(oh2, m2, 0.0)) / denom

    gu = jax.lax.dot_general(x, gup_ref[0], (((1,), (1,)), ((), ())),
                             preferred_element_type=jnp.float32)
    g = gu[:, :_I]
    u = gu[:, _I:]
    h = g * jax.lax.logistic(g) * u
    y = jax.lax.dot_general(h, down_ref[0], (((1,), (1,)), ((), ())),
                            preferred_element_type=jnp.float32)
    w_col = jnp.sum(jnp.where(iota_e == e, w_te, 0.0), axis=-1, keepdims=True)
    routed = y * w_col

    @pl.when(e == 0)
    def _():
        gs = jax.lax.dot_general(x, gw_ref[...], (((1,), (1,)), ((), ())),
                                 preferred_element_type=jnp.float32)
        us = jax.lax.dot_general(x, uw_ref[...], (((1,), (1,)), ((), ())),
                                 preferred_element_type=jnp.float32)
        hs = gs * jax.lax.logistic(gs) * us
        sh = jax.lax.dot_general(hs, dw_ref[...], (((1,), (1,)), ((), ())),
                                 preferred_element_type=jnp.float32)
        sgate = jax.lax.logistic(
            jax.lax.dot_general(x, sg_ref[...], (((1,), (1,)), ((), ())),
                                preferred_element_type=jnp.float32))
        acc_ref[pl.ds(tb * _TB, _TB), :] = sgate * sh + routed

    @pl.when(e != 0)
    def _():
        acc_ref[pl.ds(tb * _TB, _TB), :] += routed

    out_ref[...] = acc_ref[pl.ds(tb * _TB, _TB), :]


@jax.jit
def kernel(hidden_states, router_weight, gate_up_proj, down_proj,
           gate_w, up_w, down_w, shared_gate_w):
    s, b, d = hidden_states.shape
    x = hidden_states.reshape(-1, d)

    out, probs = pl.pallas_call(
        _moe_body,
        grid=(_E, _NTB),
        in_specs=[
            pl.BlockSpec((_TB, _D), lambda e, tb: (tb, 0)),
            pl.BlockSpec((_E, _D), lambda e, tb: (0, 0)),
            pl.BlockSpec((1, 2 * _I, _D), lambda e, tb: (e, 0, 0)),
            pl.BlockSpec((1, _D, _I), lambda e, tb: (e, 0, 0)),
            pl.BlockSpec((_SI, _D), lambda e, tb: (0, 0)),
            pl.BlockSpec((_SI, _D), lambda e, tb: (0, 0)),
            pl.BlockSpec((_D, _SI), lambda e, tb: (0, 0)),
            pl.BlockSpec((1, _D), lambda e, tb: (0, 0)),
        ],
        out_specs=[
            pl.BlockSpec((_TB, _D),
                         lambda e, tb: (jnp.where(e == _E - 1, tb, 0), 0)),
            pl.BlockSpec((_TB, _E),
                         lambda e, tb: (jnp.where(e == _E - 1, tb, 0), 0)),
        ],
        out_shape=[
            jax.ShapeDtypeStruct((_T, _D), jnp.float32),
            jax.ShapeDtypeStruct((_T, _E), jnp.float32),
        ],
        scratch_shapes=[pltpu.VMEM((_T, _D), jnp.float32)],
    )(x, router_weight, gate_up_proj, down_proj, gate_w, up_w, down_w,
      shared_gate_w)

    return out.reshape(s, b, d), probs
